# Initial kernel scaffold; baseline (speedup 1.0000x reference)
import jax
import jax.numpy as jnp
from jax.experimental import pallas as pl


def kernel(x, mask, hW1, hb1, hW2, hb2, hW3, hb3, rW1, rb1, rW2, rb2, rW3, rb3):
    raise NotImplementedError("write your pallas kernel here")


# speedup vs baseline: 5.6676x; 5.6676x over previous
import functools

import jax
import jax.numpy as jnp
from jax.experimental import pallas as pl

_N, _D = 16384, 50
_ROWS = 1024


def _body(x_ref, m_ref, w0_ref, B_ref, W2_ref, b2_ref, W3_ref, b3_ref,
          rW1_ref, rb1_ref, rW2_ref, rb2_ref, rW3_ref, rb3_ref,
          mu_ref, sig_ref):
    w0 = w0_ref[:]
    W2 = W2_ref[:]
    b2 = b2_ref[:]

    x = x_ref[:]
    m = m_ref[:]
    s2 = jnp.zeros((_ROWS, 64), jnp.float32)
    for d in range(_D):
        xd = x[:, d:d + 1]
        bd = B_ref[d:d + 1, :]
        h1 = jnp.maximum(xd * w0 + bd, 0.0)
        h2 = jnp.maximum(
            jnp.dot(h1, W2, preferred_element_type=jnp.float32) + b2, 0.0)
        md = m[:, d:d + 1]
        s2 = s2 + md * h2

    cnt = jnp.sum(m_ref[:], axis=1, keepdims=True)
    pooled = (jnp.dot(s2, W3_ref[:], preferred_element_type=jnp.float32)
              + cnt * b3_ref[:])
    r = jnp.maximum(
        jnp.dot(pooled, rW1_ref[:], preferred_element_type=jnp.float32)
        + rb1_ref[:], 0.0)
    r = jnp.maximum(
        jnp.dot(r, rW2_ref[:], preferred_element_type=jnp.float32)
        + rb2_ref[:], 0.0)
    g = (jnp.dot(r, rW3_ref[:], preferred_element_type=jnp.float32)
         + rb3_ref[:])
    mu_ref[:] = g[:, :64]
    sig_ref[:] = jnp.logaddexp(g[:, 64:], 0.0)


@functools.partial(jax.jit, static_argnames=("interpret",))
def _run(x, maskf, w0, B, hW2, hb2, hW3, hb3, rW1, rb1, rW2, rb2, rW3, rb3,
         interpret=False):
    grid = (_N // _ROWS,)
    row_spec = pl.BlockSpec((_ROWS, _D), lambda i: (i, 0))
    out_spec = pl.BlockSpec((_ROWS, 64), lambda i: (i, 0))

    def rep(shape):
        return pl.BlockSpec(shape, lambda i: tuple(0 for _ in shape))

    mu, sig = pl.pallas_call(
        _body,
        grid=grid,
        in_specs=[
            row_spec, row_spec,
            rep((1, 64)), rep((_D, 64)),
            rep((64, 64)), rep((1, 64)),
            rep((64, 64)), rep((1, 64)),
            rep((64, 64)), rep((1, 64)),
            rep((64, 64)), rep((1, 64)),
            rep((64, 128)), rep((1, 128)),
        ],
        out_specs=[out_spec, out_spec],
        out_shape=[
            jax.ShapeDtypeStruct((_N, 64), jnp.float32),
            jax.ShapeDtypeStruct((_N, 64), jnp.float32),
        ],
        interpret=interpret,
    )(x, maskf, w0, B, hW2, hb2, hW3, hb3, rW1, rb1, rW2, rb2, rW3, rb3)
    return mu, sig


def kernel(x, mask, hW1, hb1, hW2, hb2, hW3, hb3,
           rW1, rb1, rW2, rb2, rW3, rb3):
    maskf = mask.astype(jnp.float32)
    w0 = hW1[0:1, :]
    dim_ids = jnp.arange(_D, dtype=jnp.float32)[:, None]
    B = dim_ids * hW1[1:2, :] + hb1[None, :]
    return _run(x, maskf, w0, B, hW2, hb2[None, :], hW3, hb3[None, :],
                rW1, rb1[None, :], rW2, rb2[None, :], rW3, rb3[None, :])
